# baseline (speedup 1.0000x reference)
import jax
import jax.numpy as jnp
from jax import lax
from jax.experimental import pallas as pl
from jax.experimental.pallas import tpu as pltpu
from jax.experimental.pallas import tpu_sc as plsc

VOCAB = 1000
EMBED_DIM = 16
B = 4096
L = 20
BB = 128

_NC = 2
_NS = 16
_NW = _NC * _NS
_ROWS_W = B // _NW
_RPC = 4
_CH = _RPC * L
_NCHUNK = _ROWS_W // _RPC
_TOK_W = _ROWS_W * L


def _sc_pool_body(ids_hbm, embed_hbm, out_hbm, idx_v, rows_v, out_v, sem):
    wid = lax.axis_index("s") * _NC + lax.axis_index("c")
    base_chunk = wid * _NCHUNK
    pltpu.sync_copy(ids_hbm.at[pl.ds(base_chunk, _NCHUNK)], idx_v)

    copies = []
    for i in range(_NCHUNK):
        copies.append(
            pltpu.async_copy(embed_hbm.at[idx_v.at[i]], rows_v.at[i], sem))
    for c in copies:
        c.wait()

    def chunk(i, _):
        for j in range(_RPC):
            acc = rows_v[i, j * L, :]
            for l in range(1, L):
                acc = acc + rows_v[i, j * L + l, :]
            r = i * _RPC + j
            out_v[pl.ds(r * EMBED_DIM, EMBED_DIM)] = acc
        return _

    lax.fori_loop(0, _NCHUNK, chunk, None)
    pltpu.sync_copy(
        out_v,
        out_hbm.at[pl.ds(wid * _ROWS_W * EMBED_DIM, _ROWS_W * EMBED_DIM)])


def _sc_pool(ids2d, embed):
    mesh = plsc.VectorSubcoreMesh(core_axis_name="c", subcore_axis_name="s")
    return pl.kernel(
        _sc_pool_body,
        mesh=mesh,
        out_type=jax.ShapeDtypeStruct((B * EMBED_DIM,), jnp.float32),
        scratch_types=[
            pltpu.VMEM((_NCHUNK, _CH), jnp.int32),
            pltpu.VMEM((_NCHUNK, _CH, EMBED_DIM), jnp.float32),
            pltpu.VMEM((_ROWS_W * EMBED_DIM,), jnp.float32),
            pltpu.SemaphoreType.DMA,
        ],
        compiler_params=pltpu.CompilerParams(use_tc_tiling_on_sc=False),
    )(ids2d, embed)


def _tc_body(pooled_ref, W_ref, b_ref, out_ref):
    logits = jnp.dot(pooled_ref[...], W_ref[...], preferred_element_type=jnp.float32)
    logits = logits * (1.0 / L) + b_ref[...]
    out_ref[...] = jnp.broadcast_to(logits[:, None, :], (BB, L, VOCAB))


def kernel(input_ids, embed, W, b):
    ids2d = input_ids.reshape(B * L // _CH, _CH)
    pooled = _sc_pool(ids2d, embed).reshape(B, EMBED_DIM)
    b2 = b.reshape(1, VOCAB)
    out = pl.pallas_call(
        _tc_body,
        grid=(B // BB,),
        in_specs=[
            pl.BlockSpec((BB, EMBED_DIM), lambda i: (i, 0)),
            pl.BlockSpec((EMBED_DIM, VOCAB), lambda i: (0, 0)),
            pl.BlockSpec((1, VOCAB), lambda i: (0, 0)),
        ],
        out_specs=pl.BlockSpec((BB, L, VOCAB), lambda i: (i, 0, 0)),
        out_shape=jax.ShapeDtypeStruct((B, L, VOCAB), jnp.float32),
    )(pooled, W, b2)
    return out
